# baseline (speedup 1.0000x reference)
import functools

import jax
import jax.numpy as jnp
from jax import lax
from jax.experimental import pallas as pl
from jax.experimental.pallas import tpu as pltpu
from jax.experimental.pallas import tpu_sc as plsc

NC = 2
NS = 16
L = 16
G = 128

F32 = jnp.float32
I32 = jnp.int32


def _round_up(a, m):
  return (a + m - 1) // m * m


def _make_deg(E, NPAD):
  EC = E // (NC * NS)
  ZC = NPAD // NS
  mesh = plsc.VectorSubcoreMesh(core_axis_name="c", subcore_axis_name="s")

  @functools.partial(
      pl.kernel,
      mesh=mesh,
      out_type=jax.ShapeDtypeStruct((NC * NPAD,), F32),
      scratch_types=[
          pltpu.VMEM((EC,), F32),
          pltpu.VMEM((EC,), I32),
          pltpu.VMEM((ZC,), F32),
          pltpu.VMEM_SHARED((NPAD,), F32),
      ],
  )
  def deg_kernel(col_hbm, deg_hbm, onesv, idxv, zv, deg_sh):
    c = lax.axis_index("c")
    s = lax.axis_index("s")
    wid = c * NS + s

    one16 = jnp.full((L,), 1.0, F32)
    zero16 = jnp.zeros((L,), F32)

    def fill_ones(i, _):
      onesv[pl.ds(i * L, L)] = one16
      return 0
    lax.fori_loop(0, EC // L, fill_ones, 0)
    if EC % L:
      onesv[pl.ds(EC - L, L)] = one16

    def fill_zero(i, _):
      zv[pl.ds(i * L, L)] = zero16
      return 0
    lax.fori_loop(0, ZC // L, fill_zero, 0)

    pltpu.sync_copy(zv, deg_sh.at[pl.ds(s * ZC, ZC)])
    plsc.subcore_barrier()

    pltpu.sync_copy(col_hbm.at[pl.ds(wid * EC, EC)], idxv)
    pltpu.sync_copy(onesv, deg_sh.at[idxv], add=True)
    plsc.subcore_barrier()

    pltpu.sync_copy(deg_sh.at[pl.ds(s * ZC, ZC)], zv)
    pltpu.sync_copy(zv, deg_hbm.at[pl.ds(c * NPAD + s * ZC, ZC)])

  return deg_kernel


def _make_linear(N, CIN, COUT, NPAD, BN):
  H = COUT // 2
  grid = ((N + BN - 1) // BN,)

  def lin_kernel(x_ref, wt_ref, b_ref, dg_ref, tab_ref, dis_ref):
    h = jnp.dot(x_ref[...], wt_ref[...], preferred_element_type=F32)
    h = h + b_ref[...]
    dis = lax.rsqrt(1.0 + dg_ref[0] + dg_ref[1])
    h = h * dis[:, None]
    for half in range(2):
      tab_ref[half] = h[:, half * H:(half + 1) * H]
    dis_ref[...] = dis[None, :]

  return pl.pallas_call(
      lin_kernel,
      grid=grid,
      in_specs=[
          pl.BlockSpec((BN, CIN), lambda i: (i, 0)),
          pl.BlockSpec((CIN, COUT), lambda i: (0, 0)),
          pl.BlockSpec((1, COUT), lambda i: (0, 0)),
          pl.BlockSpec((NC, BN), lambda i: (0, i)),
      ],
      out_specs=[
          pl.BlockSpec((NC, BN, H), lambda i: (0, i, 0)),
          pl.BlockSpec((1, BN), lambda i: (0, i)),
      ],
      out_shape=[
          jax.ShapeDtypeStruct((NC, N, H), F32),
          jax.ShapeDtypeStruct((1, NPAD), F32),
      ],
  )


def _make_scatter(N, COUT, HQ, NPAD, NBT):
  HH = COUT // 2
  NB = NBT // NS
  ZR = NPAD // NS
  ZB = ZR // G
  FB = (N // G) * G
  BR = N % G
  NBUF = 2
  D = 1
  mesh = plsc.VectorSubcoreMesh(core_axis_name="c", subcore_axis_name="s")

  @functools.partial(
      pl.kernel,
      mesh=mesh,
      out_type=[
          jax.ShapeDtypeStruct((N, COUT), F32),
          jax.ShapeDtypeStruct((NC, NBT * G, HQ), F32),
      ],
      compiler_params=pltpu.CompilerParams(
          use_tc_tiling_on_sc=False, needs_layout_passes=False),
      scratch_types=[
          pltpu.VMEM((NB, G), I32),
          pltpu.VMEM((NB, G), I32),
          [pltpu.VMEM((G, HH), F32)] * NBUF,
          [pltpu.VMEM((G, HQ), F32)] * 2,
          [pltpu.VMEM((G, HQ), F32)] * 2,
          pltpu.VMEM((ZR,), F32),
          pltpu.VMEM_SHARED((NPAD, HQ), F32),
          [pltpu.SemaphoreType.DMA] * NBUF,
          [pltpu.SemaphoreType.DMA] * 2,
          [pltpu.SemaphoreType.DMA] * 2,
      ],
  )
  def scat_kernel(tab_hbm, row_hbm, colb_hbm, dis_hbm, out_hbm, spill_hbm,
                  rowv, colv, bufs, sbufs, rbufs, disv, acc_sh,
                  gsems, ssems, wsems):
    c = lax.axis_index("c")
    s = lax.axis_index("s")

    zero16 = jnp.zeros((L,), F32)

    pltpu.sync_copy(row_hbm.at[pl.ds(s * NB, NB), :], rowv)
    pltpu.sync_copy(colb_hbm.at[c, pl.ds(s * NB, NB), :], colv)
    pltpu.sync_copy(dis_hbm.at[pl.ds(s * ZR, ZR)], disv)

    def spill_slice(j):
      return spill_hbm.at[c, pl.ds((s * NB + j) * G, G), :]

    def gather(j, k):
      pltpu.async_copy(tab_hbm.at[colv.at[j]], bufs[k], gsems[k])

    def gather_wait(j, k):
      pltpu.make_async_copy(tab_hbm.at[colv.at[j]], bufs[k], gsems[k]).wait()

    def split(k, k2):
      def srow(r, _):
        for kk in range(HQ // L):
          sbufs[k2][r, pl.ds(kk * L, L)] = bufs[k][r, pl.ds(kk * L, L)]
          rbufs[k2][r, pl.ds(kk * L, L)] = bufs[k][r, pl.ds(HQ + kk * L, L)]
        return 0
      lax.fori_loop(0, G, srow, 0)

    def scat(j, k2):
      pltpu.async_copy(sbufs[k2], acc_sh.at[rowv.at[j]], ssems[k2], add=True)

    def scat_wait(j, k2):
      pltpu.make_async_copy(sbufs[k2], acc_sh.at[rowv.at[j]],
                            ssems[k2]).wait()

    def spill(j, k2):
      pltpu.async_copy(rbufs[k2], spill_slice(j), wsems[k2])

    def spill_wait(j, k2):
      pltpu.make_async_copy(rbufs[k2], spill_slice(j), wsems[k2]).wait()

    def replay_read(j, k2):
      pltpu.async_copy(spill_slice(j), rbufs[k2], wsems[k2])

    def replay_read_wait(j, k2):
      pltpu.make_async_copy(spill_slice(j), rbufs[k2], wsems[k2]).wait()

    def rscat(j, k2):
      pltpu.async_copy(rbufs[k2], acc_sh.at[rowv.at[j]], ssems[k2], add=True)

    def rscat_wait(j, k2):
      pltpu.make_async_copy(rbufs[k2], acc_sh.at[rowv.at[j]],
                            ssems[k2]).wait()

    def zero_acc():
      def zrow(i, _):
        for kk in range(HQ // L):
          rbufs[0][i, pl.ds(kk * L, L)] = zero16
        return 0
      lax.fori_loop(0, G, zrow, 0)
      for m in range(ZB):
        pltpu.sync_copy(rbufs[0], acc_sh.at[pl.ds(s * ZR + m * G, G), :])

    def finalize(p, rs, rc):
      q = c * 2 + p
      pltpu.sync_copy(acc_sh.at[pl.ds(rs, rc), :], rbufs[0].at[pl.ds(0, rc), :])
      pltpu.sync_copy(tab_hbm.at[pl.ds(c * N + rs, rc), :],
                      bufs[0].at[pl.ds(0, rc), :])

      def frow(r, _):
        dvec = plsc.load_gather(disv, [jnp.full((L,), rs - s * ZR + r, I32)])
        for kk in range(HQ // L):
          rbufs[0][r, pl.ds(kk * L, L)] = (
              rbufs[0][r, pl.ds(kk * L, L)]
              + bufs[0][r, pl.ds(p * HQ + kk * L, L)]) * dvec
        return 0

      lax.fori_loop(0, rc, frow, 0)
      pltpu.sync_copy(rbufs[0].at[pl.ds(0, rc), :],
                      out_hbm.at[pl.ds(rs, rc), pl.ds(q * HQ, HQ)])

    def finalize_all(p):
      for m in range(ZB):
        rs = s * ZR + m * G

        @pl.when(rs + G <= N)
        def _():
          finalize(p, rs, G)
        if BR:
          @pl.when(rs == FB)
          def _():
            finalize(p, rs, BR)

    zero_acc()
    plsc.subcore_barrier()

    for j in range(D):
      gather(j, j % NBUF)

    def body(r, _):
      for k in range(2):
        j = r * 2 + k
        k2 = k

        @pl.when(j + 1 < NB)
        def _():
          gather(j + 1, (k + 1) % 2)

        gather_wait(j, k)

        @pl.when(j - 2 >= 0)
        def _():
          scat_wait(j - 2, k2)
          spill_wait(j - 2, k2)

        split(k, k2)
        scat(j, k2)
        spill(j, k2)
      return 0

    lax.fori_loop(0, NB // 2, body, 0)
    for j in range(NB - 2, NB):
      scat_wait(j, j % 2)
      spill_wait(j, j % 2)
    plsc.subcore_barrier()
    finalize_all(0)

    zero_acc()
    plsc.subcore_barrier()

    replay_read(0, 0)

    def body2(r, _):
      for k in range(2):
        j = r * 2 + k
        k2 = k
        k2o = (k + 1) % 2

        @pl.when(j + 1 < NB)
        def _():
          @pl.when(j - 1 >= 0)
          def _():
            rscat_wait(j - 1, k2o)
          replay_read(j + 1, k2o)

        replay_read_wait(j, k2)
        rscat(j, k2)
      return 0

    lax.fori_loop(0, NB // 2, body2, 0)
    for j in range(NB - 2, NB):
      rscat_wait(j, j % 2)
    plsc.subcore_barrier()
    finalize_all(1)

  return scat_kernel


@jax.jit
def kernel(x, edge_index, W, b):
  N, CIN = x.shape
  COUT = W.shape[0]
  H = COUT // 4
  E = edge_index.shape[1]

  NPAD = _round_up(N + 1, NS * G)
  E2 = _round_up(E, NS * G * 8)
  NBT = E2 // G
  BN = 512

  row = edge_index[0]
  col = edge_index[1]
  pad = E2 - E
  rowp = jnp.concatenate([row, jnp.full((pad,), N, I32)]).reshape(NBT, G)
  colp = jnp.concatenate([col, jnp.zeros((pad,), I32)]).reshape(NBT, G)
  colb = jnp.stack([colp, colp + N])

  degsum = _make_deg(E, NPAD)(col).reshape(NC, NPAD)
  table, dis = _make_linear(N, CIN, COUT, NPAD, BN)(
      x, W.T, b[None, :], degsum)
  out, _ = _make_scatter(N, COUT, H, NPAD, NBT)(
      table.reshape(NC * N, COUT // 2), rowp, colb, dis.reshape(NPAD))
  return out


# speedup vs baseline: 1.8529x; 1.8529x over previous
import functools

import jax
import jax.numpy as jnp
from jax import lax
from jax.experimental import pallas as pl
from jax.experimental.pallas import tpu as pltpu
from jax.experimental.pallas import tpu_sc as plsc

NC = 2
NS = 16
L = 16
G = 128

F32 = jnp.float32
I32 = jnp.int32
MHI = -65536


def _round_up(a, m):
  return (a + m - 1) // m * m


def _make_deg(E, NPAD):
  EC = E // (NC * NS)
  ZC = NPAD // NS
  mesh = plsc.VectorSubcoreMesh(core_axis_name="c", subcore_axis_name="s")

  @functools.partial(
      pl.kernel,
      mesh=mesh,
      out_type=jax.ShapeDtypeStruct((NC * NPAD,), F32),
      scratch_types=[
          pltpu.VMEM((EC,), F32),
          pltpu.VMEM((EC,), I32),
          pltpu.VMEM((ZC,), F32),
          pltpu.VMEM_SHARED((NPAD,), F32),
      ],
  )
  def deg_kernel(col_hbm, deg_hbm, onesv, idxv, zv, deg_sh):
    c = lax.axis_index("c")
    s = lax.axis_index("s")
    wid = c * NS + s

    one16 = jnp.full((L,), 1.0, F32)
    zero16 = jnp.zeros((L,), F32)

    def fill_ones(i, _):
      onesv[pl.ds(i * L, L)] = one16
      return 0
    lax.fori_loop(0, EC // L, fill_ones, 0)
    if EC % L:
      onesv[pl.ds(EC - L, L)] = one16

    def fill_zero(i, _):
      zv[pl.ds(i * L, L)] = zero16
      return 0
    lax.fori_loop(0, ZC // L, fill_zero, 0)

    pltpu.sync_copy(zv, deg_sh.at[pl.ds(s * ZC, ZC)])
    plsc.subcore_barrier()

    pltpu.sync_copy(col_hbm.at[pl.ds(wid * EC, EC)], idxv)
    pltpu.sync_copy(onesv, deg_sh.at[idxv], add=True)
    plsc.subcore_barrier()

    pltpu.sync_copy(deg_sh.at[pl.ds(s * ZC, ZC)], zv)
    pltpu.sync_copy(zv, deg_hbm.at[pl.ds(c * NPAD + s * ZC, ZC)])

  return deg_kernel


def _make_linear(N, CIN, COUT, NPAD, BN):
  HQ = COUT // 4
  HP = HQ // 2
  grid = ((N + BN - 1) // BN,)

  def lin_kernel(x_ref, wt_ref, b_ref, dg_ref, tab_ref, dis_ref):
    h = jnp.dot(x_ref[...], wt_ref[...], preferred_element_type=F32)
    h = h + b_ref[...]
    dis = lax.rsqrt(1.0 + dg_ref[0] + dg_ref[1])
    h = h * dis[:, None]
    for q in range(4):
      lo = h[:, q * HQ:q * HQ + HP]
      hi = h[:, q * HQ + HP:(q + 1) * HQ]
      lo16 = lax.bitcast_convert_type(
          lo.astype(jnp.bfloat16), jnp.uint16).astype(jnp.uint32)
      hi16 = lax.bitcast_convert_type(
          hi.astype(jnp.bfloat16), jnp.uint16).astype(jnp.uint32)
      w = lo16 | (hi16 << jnp.uint32(16))
      tab_ref[q] = lax.bitcast_convert_type(w, I32)
    dis_ref[...] = dis[None, :]

  return pl.pallas_call(
      lin_kernel,
      grid=grid,
      in_specs=[
          pl.BlockSpec((BN, CIN), lambda i: (i, 0)),
          pl.BlockSpec((CIN, COUT), lambda i: (0, 0)),
          pl.BlockSpec((1, COUT), lambda i: (0, 0)),
          pl.BlockSpec((NC, BN), lambda i: (0, i)),
      ],
      out_specs=[
          pl.BlockSpec((4, BN, HP), lambda i: (0, i, 0)),
          pl.BlockSpec((1, BN), lambda i: (0, i)),
      ],
      out_shape=[
          jax.ShapeDtypeStruct((4, N, HP), I32),
          jax.ShapeDtypeStruct((1, NPAD), F32),
      ],
  )


def _make_scatter(N, COUT, NPAD, NBT):
  HQ = COUT // 4
  HP = HQ // 2
  NB = NBT // NS
  ZR = NPAD // NS
  ZB = ZR // G
  FB = (N // G) * G
  BR = N % G
  mesh = plsc.VectorSubcoreMesh(core_axis_name="c", subcore_axis_name="s")

  @functools.partial(
      pl.kernel,
      mesh=mesh,
      out_type=jax.ShapeDtypeStruct((N, COUT), F32),
      compiler_params=pltpu.CompilerParams(
          use_tc_tiling_on_sc=False, needs_layout_passes=False),
      scratch_types=[
          pltpu.VMEM((NB, G), I32),
          pltpu.VMEM((NB, G), I32),
          [pltpu.VMEM((G, HP), I32)] * 4,
          [pltpu.VMEM((G, HQ), F32)] * 2,
          pltpu.VMEM((ZR,), F32),
          pltpu.VMEM_SHARED((NPAD, HQ), F32),
          [pltpu.SemaphoreType.DMA] * 4,
          [pltpu.SemaphoreType.DMA] * 2,
      ],
  )
  def scat_kernel(tab_hbm, row_hbm, colb_hbm, dis_hbm, out_hbm,
                  rowv, colv, gbufs, fbufs, disv, acc_sh, gsems, ssems):
    c = lax.axis_index("c")
    s = lax.axis_index("s")

    zero16 = jnp.zeros((L,), F32)

    pltpu.sync_copy(row_hbm.at[pl.ds(s * NB, NB), :], rowv)
    pltpu.sync_copy(dis_hbm.at[pl.ds(s * ZR, ZR)], disv)

    def gather(j, k):
      pltpu.async_copy(tab_hbm.at[colv.at[j]], gbufs[k], gsems[k])

    def gather_wait(j, k):
      pltpu.make_async_copy(tab_hbm.at[colv.at[j]], gbufs[k], gsems[k]).wait()

    def scat(j, k2):
      pltpu.async_copy(fbufs[k2], acc_sh.at[rowv.at[j]], ssems[k2], add=True)

    def scat_wait(j, k2):
      pltpu.make_async_copy(fbufs[k2], acc_sh.at[rowv.at[j]],
                            ssems[k2]).wait()

    def unpack_row(src, r):
      w0 = src[r, pl.ds(0, L)]
      w1 = src[r, pl.ds(L, L)]
      return (plsc.bitcast(w0 << 16, F32),
              plsc.bitcast(w1 << 16, F32),
              plsc.bitcast(w0 & MHI, F32),
              plsc.bitcast(w1 & MHI, F32))

    def convert(k, k2):
      def crow(r, _):
        f0, f1, f2, f3 = unpack_row(gbufs[k], r)
        fbufs[k2][r, pl.ds(0, L)] = f0
        fbufs[k2][r, pl.ds(L, L)] = f1
        fbufs[k2][r, pl.ds(2 * L, L)] = f2
        fbufs[k2][r, pl.ds(3 * L, L)] = f3
        return 0
      lax.fori_loop(0, G, crow, 0)

    def zero_acc():
      def zrow(i, _):
        for kk in range(HQ // L):
          fbufs[0][i, pl.ds(kk * L, L)] = zero16
        return 0
      lax.fori_loop(0, G, zrow, 0)
      for m in range(ZB):
        pltpu.sync_copy(fbufs[0], acc_sh.at[pl.ds(s * ZR + m * G, G), :])

    for p in range(2):
      q = c * 2 + p

      zero_acc()
      pltpu.sync_copy(colb_hbm.at[q, pl.ds(s * NB, NB), :], colv)
      plsc.subcore_barrier()

      gather(0, 0)
      gather(1, 1)

      def body(r, _):
        for k in range(4):
          j = r * 4 + k
          k2 = k % 2

          @pl.when(j + 2 < NB)
          def _():
            gather(j + 2, (k + 2) % 4)

          gather_wait(j, k)

          @pl.when(j - 2 >= 0)
          def _():
            scat_wait(j - 2, k2)

          convert(k, k2)
          scat(j, k2)
        return 0

      lax.fori_loop(0, NB // 4, body, 0)
      for j in range(NB - 2, NB):
        scat_wait(j, j % 2)
      plsc.subcore_barrier()

      def finalize(rs, rc):
        pltpu.sync_copy(acc_sh.at[pl.ds(rs, rc), :],
                        fbufs[0].at[pl.ds(0, rc), :])
        pltpu.sync_copy(tab_hbm.at[pl.ds(q * N + rs, rc), :],
                        gbufs[0].at[pl.ds(0, rc), :])

        def frow(r, _):
          dvec = plsc.load_gather(disv, [jnp.full((L,), rs - s * ZR + r, I32)])
          t0, t1, t2, t3 = unpack_row(gbufs[0], r)
          for kk, t in enumerate((t0, t1, t2, t3)):
            sl = pl.ds(kk * L, L)
            fbufs[0][r, sl] = (fbufs[0][r, sl] + t) * dvec
          return 0

        lax.fori_loop(0, rc, frow, 0)
        pltpu.sync_copy(fbufs[0].at[pl.ds(0, rc), :],
                        out_hbm.at[pl.ds(rs, rc), pl.ds(q * HQ, HQ)])

      for m in range(ZB):
        rs = s * ZR + m * G

        @pl.when(rs + G <= N)
        def _():
          finalize(rs, G)
        if BR:
          @pl.when(rs == FB)
          def _():
            finalize(rs, BR)

  return scat_kernel


@jax.jit
def kernel(x, edge_index, W, b):
  N, CIN = x.shape
  COUT = W.shape[0]
  E = edge_index.shape[1]

  NPAD = _round_up(N + 1, NS * G)
  E2 = _round_up(E, NS * G * 8)
  NBT = E2 // G
  BN = 512

  row = edge_index[0]
  col = edge_index[1]
  pad = E2 - E
  rowp = jnp.concatenate([row, jnp.full((pad,), N, I32)]).reshape(NBT, G)
  colp = jnp.concatenate([col, jnp.zeros((pad,), I32)]).reshape(NBT, G)
  colb = jnp.stack([colp + q * N for q in range(4)])

  degsum = _make_deg(E, NPAD)(col).reshape(NC, NPAD)
  table, dis = _make_linear(N, CIN, COUT, NPAD, BN)(
      x, W.T, b[None, :], degsum)
  out = _make_scatter(N, COUT, NPAD, NBT)(
      table.reshape(4 * N, COUT // 8), rowp, colb, dis.reshape(NPAD))
  return out


# speedup vs baseline: 1.8703x; 1.0094x over previous
import functools

import jax
import jax.numpy as jnp
from jax import lax
from jax.experimental import pallas as pl
from jax.experimental.pallas import tpu as pltpu
from jax.experimental.pallas import tpu_sc as plsc

NC = 2
NS = 16
L = 16
G = 128

F32 = jnp.float32
I32 = jnp.int32
MHI = -65536


def _round_up(a, m):
  return (a + m - 1) // m * m


def _make_deg(E, NPAD):
  EC = E // (NC * NS)
  ZC = NPAD // NS
  mesh = plsc.VectorSubcoreMesh(core_axis_name="c", subcore_axis_name="s")

  @functools.partial(
      pl.kernel,
      mesh=mesh,
      out_type=jax.ShapeDtypeStruct((NC * NPAD,), F32),
      scratch_types=[
          pltpu.VMEM((EC,), F32),
          pltpu.VMEM((EC,), I32),
          pltpu.VMEM((ZC,), F32),
          pltpu.VMEM_SHARED((NPAD,), F32),
      ],
  )
  def deg_kernel(col_hbm, deg_hbm, onesv, idxv, zv, deg_sh):
    c = lax.axis_index("c")
    s = lax.axis_index("s")
    wid = c * NS + s

    one16 = jnp.full((L,), 1.0, F32)
    zero16 = jnp.zeros((L,), F32)

    def fill_ones(i, _):
      onesv[pl.ds(i * L, L)] = one16
      return 0
    lax.fori_loop(0, EC // L, fill_ones, 0)
    if EC % L:
      onesv[pl.ds(EC - L, L)] = one16

    def fill_zero(i, _):
      zv[pl.ds(i * L, L)] = zero16
      return 0
    lax.fori_loop(0, ZC // L, fill_zero, 0)

    pltpu.sync_copy(zv, deg_sh.at[pl.ds(s * ZC, ZC)])
    plsc.subcore_barrier()

    pltpu.sync_copy(col_hbm.at[pl.ds(wid * EC, EC)], idxv)
    pltpu.sync_copy(onesv, deg_sh.at[idxv], add=True)
    plsc.subcore_barrier()

    pltpu.sync_copy(deg_sh.at[pl.ds(s * ZC, ZC)], zv)
    pltpu.sync_copy(zv, deg_hbm.at[pl.ds(c * NPAD + s * ZC, ZC)])

  return deg_kernel


def _make_linear(N, CIN, COUT, NPAD, BN):
  HQ = COUT // 4
  HP = HQ // 2
  grid = ((N + BN - 1) // BN,)

  def lin_kernel(x_ref, wt_ref, b_ref, dg_ref, tab_ref, dis_ref):
    h = jnp.dot(x_ref[...], wt_ref[...], preferred_element_type=F32)
    h = h + b_ref[...]
    dis = lax.rsqrt(1.0 + dg_ref[0] + dg_ref[1])
    h = h * dis[:, None]
    for q in range(4):
      lo = h[:, q * HQ:q * HQ + HP]
      hi = h[:, q * HQ + HP:(q + 1) * HQ]
      lo16 = lax.bitcast_convert_type(
          lo.astype(jnp.bfloat16), jnp.uint16).astype(jnp.uint32)
      hi16 = lax.bitcast_convert_type(
          hi.astype(jnp.bfloat16), jnp.uint16).astype(jnp.uint32)
      w = lo16 | (hi16 << jnp.uint32(16))
      tab_ref[q] = lax.bitcast_convert_type(w, I32)
    dis_ref[...] = dis[None, :]

  return pl.pallas_call(
      lin_kernel,
      grid=grid,
      in_specs=[
          pl.BlockSpec((BN, CIN), lambda i: (i, 0)),
          pl.BlockSpec((CIN, COUT), lambda i: (0, 0)),
          pl.BlockSpec((1, COUT), lambda i: (0, 0)),
          pl.BlockSpec((NC, BN), lambda i: (0, i)),
      ],
      out_specs=[
          pl.BlockSpec((4, BN, HP), lambda i: (0, i, 0)),
          pl.BlockSpec((1, BN), lambda i: (0, i)),
      ],
      out_shape=[
          jax.ShapeDtypeStruct((4, N, HP), I32),
          jax.ShapeDtypeStruct((1, NPAD), F32),
      ],
  )


def _make_scatter(N, COUT, NPAD, NBT):
  HQ = COUT // 4
  HP = HQ // 2
  NB = NBT // NS
  ZR = NPAD // NS
  ZB = ZR // G
  FB = (N // G) * G
  BR = N % G
  mesh = plsc.VectorSubcoreMesh(core_axis_name="c", subcore_axis_name="s")

  @functools.partial(
      pl.kernel,
      mesh=mesh,
      out_type=jax.ShapeDtypeStruct((N, COUT), F32),
      compiler_params=pltpu.CompilerParams(
          use_tc_tiling_on_sc=False, needs_layout_passes=False),
      scratch_types=[
          pltpu.VMEM((NB, G), I32),
          pltpu.VMEM((NB, G), I32),
          [pltpu.VMEM((G, HP), I32)] * 8,
          [pltpu.VMEM((G, HQ), F32)] * 2,
          pltpu.VMEM((ZR,), F32),
          pltpu.VMEM_SHARED((NPAD, HQ), F32),
          [pltpu.SemaphoreType.DMA] * 8,
          [pltpu.SemaphoreType.DMA] * 2,
      ],
  )
  def scat_kernel(tab_hbm, row_hbm, colb_hbm, dis_hbm, out_hbm,
                  rowv, colv, gbufs, fbufs, disv, acc_sh, gsems, ssems):
    c = lax.axis_index("c")
    s = lax.axis_index("s")

    zero16 = jnp.zeros((L,), F32)

    pltpu.sync_copy(row_hbm.at[pl.ds(s * NB, NB), :], rowv)
    pltpu.sync_copy(dis_hbm.at[pl.ds(s * ZR, ZR)], disv)

    def gather(j, k):
      pltpu.async_copy(tab_hbm.at[colv.at[j]], gbufs[k], gsems[k])

    def gather_wait(j, k):
      pltpu.make_async_copy(tab_hbm.at[colv.at[j]], gbufs[k], gsems[k]).wait()

    def scat(j, k2):
      pltpu.async_copy(fbufs[k2], acc_sh.at[rowv.at[j]], ssems[k2], add=True)

    def scat_wait(j, k2):
      pltpu.make_async_copy(fbufs[k2], acc_sh.at[rowv.at[j]],
                            ssems[k2]).wait()

    def unpack_row(src, r):
      w0 = src[r, pl.ds(0, L)]
      w1 = src[r, pl.ds(L, L)]
      return (plsc.bitcast(w0 << 16, F32),
              plsc.bitcast(w1 << 16, F32),
              plsc.bitcast(w0 & MHI, F32),
              plsc.bitcast(w1 & MHI, F32))

    def convert(k, k2):
      def crow(r, _):
        f0, f1, f2, f3 = unpack_row(gbufs[k], r)
        fbufs[k2][r, pl.ds(0, L)] = f0
        fbufs[k2][r, pl.ds(L, L)] = f1
        fbufs[k2][r, pl.ds(2 * L, L)] = f2
        fbufs[k2][r, pl.ds(3 * L, L)] = f3
        return 0
      lax.fori_loop(0, G, crow, 0)

    def zero_acc():
      def zrow(i, _):
        for kk in range(HQ // L):
          fbufs[0][i, pl.ds(kk * L, L)] = zero16
        return 0
      lax.fori_loop(0, G, zrow, 0)
      for m in range(ZB):
        pltpu.sync_copy(fbufs[0], acc_sh.at[pl.ds(s * ZR + m * G, G), :])

    for p in range(2):
      q = c * 2 + p

      zero_acc()
      pltpu.sync_copy(colb_hbm.at[q, pl.ds(s * NB, NB), :], colv)
      plsc.subcore_barrier()

      for jj in range(6):
        gather(jj, jj)

      def body(r, _):
        for k in range(8):
          j = r * 8 + k
          k2 = k % 2

          @pl.when(j + 6 < NB)
          def _():
            gather(j + 6, (k + 6) % 8)

          gather_wait(j, k)

          @pl.when(j - 2 >= 0)
          def _():
            scat_wait(j - 2, k2)

          convert(k, k2)
          scat(j, k2)
        return 0

      lax.fori_loop(0, NB // 8, body, 0)
      for j in range(NB - 2, NB):
        scat_wait(j, j % 2)
      plsc.subcore_barrier()

      def finalize(rs, rc):
        pltpu.sync_copy(acc_sh.at[pl.ds(rs, rc), :],
                        fbufs[0].at[pl.ds(0, rc), :])
        pltpu.sync_copy(tab_hbm.at[pl.ds(q * N + rs, rc), :],
                        gbufs[0].at[pl.ds(0, rc), :])

        def frow(r, _):
          dvec = plsc.load_gather(disv, [jnp.full((L,), rs - s * ZR + r, I32)])
          t0, t1, t2, t3 = unpack_row(gbufs[0], r)
          for kk, t in enumerate((t0, t1, t2, t3)):
            sl = pl.ds(kk * L, L)
            fbufs[0][r, sl] = (fbufs[0][r, sl] + t) * dvec
          return 0

        lax.fori_loop(0, rc, frow, 0)
        pltpu.sync_copy(fbufs[0].at[pl.ds(0, rc), :],
                        out_hbm.at[pl.ds(rs, rc), pl.ds(q * HQ, HQ)])

      for m in range(ZB):
        rs = s * ZR + m * G

        @pl.when(rs + G <= N)
        def _():
          finalize(rs, G)
        if BR:
          @pl.when(rs == FB)
          def _():
            finalize(rs, BR)

  return scat_kernel


@jax.jit
def kernel(x, edge_index, W, b):
  N, CIN = x.shape
  COUT = W.shape[0]
  E = edge_index.shape[1]

  NPAD = _round_up(N + 1, NS * G)
  E2 = _round_up(E, NS * G * 8)
  NBT = E2 // G
  BN = 512

  row = edge_index[0]
  col = edge_index[1]
  pad = E2 - E
  rowp = jnp.concatenate([row, jnp.full((pad,), N, I32)]).reshape(NBT, G)
  colp = jnp.concatenate([col, jnp.zeros((pad,), I32)]).reshape(NBT, G)
  colb = jnp.stack([colp + q * N for q in range(4)])

  degsum = _make_deg(E, NPAD)(col).reshape(NC, NPAD)
  table, dis = _make_linear(N, CIN, COUT, NPAD, BN)(
      x, W.T, b[None, :], degsum)
  out = _make_scatter(N, COUT, NPAD, NBT)(
      table.reshape(4 * N, COUT // 8), rowp, colb, dis.reshape(NPAD))
  return out
